# Initial kernel scaffold; baseline (speedup 1.0000x reference)
import jax
import jax.numpy as jnp
from jax.experimental import pallas as pl


def kernel(x, edge_index, edge_feats, W_src_1, b_src_1, W_src_2, b_src_2, W_dst_1, b_dst_1, W_dst_2, b_dst_2, W_edge_1, b_edge_1, W_edge_2, b_edge_2, W_m_1, b_m_1, W_m_2, b_m_2):
    raise NotImplementedError("write your pallas kernel here")


# speedup vs baseline: 3.0496x; 3.0496x over previous
import functools
import math

import jax
import jax.numpy as jnp
from jax import lax
from jax.experimental import pallas as pl
from jax.experimental.pallas import tpu as pltpu
from jax.experimental.pallas import tpu_sc as plsc

N = 10000
E = 320000
D = 128

NC = 2
NS = 16
NW = NC * NS

CH = 128
TOTAL_CH = E // CH
BASE_NCH = TOTAL_CH // NW
EXTRA_CH = TOTAL_CH - BASE_NCH * NW

E_CHARGE = 1.602176634e-19
EPS0 = 8.8541878128e-12
COUL = E_CHARGE * E_CHARGE / (4.0 * math.pi * EPS0 * 1e-10)

NODE_BLK = 1000
EDGE_BLK = 3200

def _softplus(v):
    return jnp.maximum(v, 0.0) + jnp.log1p(jnp.exp(-jnp.abs(v)))


def _mish(v):
    return v * jnp.tanh(_softplus(v))


def _mlp_block(v, w1, b1, w2, b2):
    t = _mish(jnp.dot(v, w1, preferred_element_type=jnp.float32) + b1)
    return jnp.dot(t, w2, preferred_element_type=jnp.float32) + b2


def _node_body(x_ref, ws1, bs1, ws2, bs2, wd1, bd1, wd2, bd2, hs_ref, hd_ref):
    xb = x_ref[...]
    hs_ref[...] = _mlp_block(xb, ws1[...], bs1[...], ws2[...], bs2[...])
    hd_ref[...] = _mlp_block(xb, wd1[...], bd1[...], wd2[...], bd2[...])


def _edge_body(ef_ref, gs_ref, gd_ref, we1, be1, we2, be2,
               wm1, bm1, wm2, bm2, m_ref):
    ef = _mlp_block(ef_ref[...], we1[...], be1[...], we2[...], be2[...])
    hn = (gs_ref[...] * gd_ref[...]) * COUL / ef
    m_ref[...] = _mlp_block(hn, wm1[...], bm1[...], wm2[...], bm2[...])


def _final_body(x_ref, hp_ref, o_ref):
    o_ref[...] = _softplus(x_ref[...] + hp_ref[0] + hp_ref[1])


def _worker_chunks():
    w = lax.axis_index("s") * NC + lax.axis_index("c")
    nch = BASE_NCH + jnp.where(w < EXTRA_CH, 1, 0)
    start = w * BASE_NCH + jnp.minimum(w, EXTRA_CH)
    return w, start, nch


def _gather_body(hs_hbm, hd_hbm, src_hbm, dst_hbm, gs_hbm, gd_hbm,
                 idx_s, idx_d, rows_s, rows_d, sem):
    _, start, nch = _worker_chunks()

    def chunk(c, carry):
        b = (start + c) * CH
        pltpu.sync_copy(src_hbm.at[pl.ds(b, CH)], idx_s)
        pltpu.sync_copy(dst_hbm.at[pl.ds(b, CH)], idx_d)
        cp1 = pltpu.async_copy(hs_hbm.at[idx_s], rows_s, sem)
        cp2 = pltpu.async_copy(hd_hbm.at[idx_d], rows_d, sem)
        cp1.wait()
        cp2.wait()
        pltpu.sync_copy(rows_s, gs_hbm.at[pl.ds(b, CH)])
        pltpu.sync_copy(rows_d, gd_hbm.at[pl.ds(b, CH)])
        return carry

    lax.fori_loop(0, nch, chunk, 0)


_BLK_R = 80
_NBLK_R = N // _BLK_R
_RB_BASE = _NBLK_R // NS
_RB_EXTRA = _NBLK_R - _RB_BASE * NS


def _scatter_body(m_hbm, dst_hbm, hp_hbm, acc, idx_v, rows_v, zrow, sem):
    cid = lax.axis_index("c")
    sid = lax.axis_index("s")
    _, start, nch = _worker_chunks()
    nblk = _RB_BASE + jnp.where(sid < _RB_EXTRA, 1, 0)

    zv = jnp.zeros((16,), jnp.float32)

    def zfill(i, carry):
        for j in range(D // 16):
            zrow[i, pl.ds(j * 16, 16)] = zv
        return carry

    lax.fori_loop(0, _BLK_R, zfill, 0)

    def zblk(c, carry):
        k = sid + NS * c
        pltpu.sync_copy(zrow, acc.at[pl.ds(k * _BLK_R, _BLK_R)])
        return carry

    lax.fori_loop(0, nblk, zblk, 0)
    plsc.subcore_barrier()

    def chunk(c, carry):
        b = (start + c) * CH
        pltpu.sync_copy(dst_hbm.at[pl.ds(b, CH)], idx_v)
        pltpu.sync_copy(m_hbm.at[pl.ds(b, CH)], rows_v)
        pltpu.sync_copy(rows_v, acc.at[idx_v], add=True)
        return carry

    lax.fori_loop(0, nch, chunk, 0)
    plsc.subcore_barrier()

    def wblk(c, carry):
        k = sid + NS * c
        pltpu.sync_copy(acc.at[pl.ds(k * _BLK_R, _BLK_R)],
                        hp_hbm.at[cid, pl.ds(k * _BLK_R, _BLK_R)])
        return carry

    lax.fori_loop(0, nblk, wblk, 0)


@functools.cache
def _sc_kernels():
    mesh = plsc.VectorSubcoreMesh(
        core_axis_name="c", subcore_axis_name="s",
        num_cores=NC, num_subcores=NS)
    gather = pl.kernel(
        _gather_body,
        out_type=(jax.ShapeDtypeStruct((E, D), jnp.float32),
                  jax.ShapeDtypeStruct((E, D), jnp.float32)),
        mesh=mesh,
        scratch_types=[
            pltpu.VMEM((CH,), jnp.int32),
            pltpu.VMEM((CH,), jnp.int32),
            pltpu.VMEM((CH, D), jnp.float32),
            pltpu.VMEM((CH, D), jnp.float32),
            pltpu.SemaphoreType.DMA,
        ],
    )
    scatter = pl.kernel(
        _scatter_body,
        out_type=jax.ShapeDtypeStruct((NC, N, D), jnp.float32),
        mesh=mesh,
        scratch_types=[
            pltpu.VMEM_SHARED((N, D), jnp.float32),
            pltpu.VMEM((CH,), jnp.int32),
            pltpu.VMEM((CH, D), jnp.float32),
            pltpu.VMEM((_BLK_R, D), jnp.float32),
            pltpu.SemaphoreType.DMA,
        ],
    )
    return gather, scatter


def _rep(i):
    return (0, 0)


def kernel(x, edge_index, edge_feats,
           W_src_1, b_src_1, W_src_2, b_src_2,
           W_dst_1, b_dst_1, W_dst_2, b_dst_2,
           W_edge_1, b_edge_1, W_edge_2, b_edge_2,
           W_m_1, b_m_1, W_m_2, b_m_2):
    src = edge_index[0]
    dst = edge_index[1]
    biases = [b.reshape(1, D) for b in
              (b_src_1, b_src_2, b_dst_1, b_dst_2,
               b_edge_1, b_edge_2, b_m_1, b_m_2)]
    bs1, bs2, bd1, bd2, be1, be2, bm1, bm2 = biases

    wspec = pl.BlockSpec((D, D), _rep)
    bspec = pl.BlockSpec((1, D), _rep)

    h_src, h_dst = pl.pallas_call(
        _node_body,
        grid=(N // NODE_BLK,),
        in_specs=[pl.BlockSpec((NODE_BLK, D), lambda i: (i, 0)),
                  wspec, bspec, wspec, bspec,
                  wspec, bspec, wspec, bspec],
        out_specs=[pl.BlockSpec((NODE_BLK, D), lambda i: (i, 0))] * 2,
        out_shape=[jax.ShapeDtypeStruct((N, D), jnp.float32)] * 2,
    )(x, W_src_1, bs1, W_src_2, bs2, W_dst_1, bd1, W_dst_2, bd2)

    gather_k, scatter_k = _sc_kernels()
    gs, gd = gather_k(h_src, h_dst, src, dst)

    m = pl.pallas_call(
        _edge_body,
        grid=(E // EDGE_BLK,),
        in_specs=[pl.BlockSpec((EDGE_BLK, D), lambda i: (i, 0))] * 3 +
                 [wspec, bspec, wspec, bspec, wspec, bspec, wspec, bspec],
        out_specs=pl.BlockSpec((EDGE_BLK, D), lambda i: (i, 0)),
        out_shape=jax.ShapeDtypeStruct((E, D), jnp.float32),
    )(edge_feats, gs, gd, W_edge_1, be1, W_edge_2, be2,
      W_m_1, bm1, W_m_2, bm2)

    hp = scatter_k(m, dst)

    out = pl.pallas_call(
        _final_body,
        grid=(N // NODE_BLK,),
        in_specs=[pl.BlockSpec((NODE_BLK, D), lambda i: (i, 0)),
                  pl.BlockSpec((NC, NODE_BLK, D), lambda i: (0, i, 0))],
        out_specs=pl.BlockSpec((NODE_BLK, D), lambda i: (i, 0)),
        out_shape=jax.ShapeDtypeStruct((N, D), jnp.float32),
    )(x, hp)
    return out


# speedup vs baseline: 4.4137x; 1.4473x over previous
import functools
import math

import jax
import jax.numpy as jnp
from jax import lax
from jax.experimental import pallas as pl
from jax.experimental.pallas import tpu as pltpu
from jax.experimental.pallas import tpu_sc as plsc

N = 10000
E = 320000
D = 128

NC = 2
NS = 16
NW = NC * NS

CH = 80
W_EDGES = E // NW
W_CH = W_EDGES // CH
W_PAIRS = (W_CH - 1) // 2

E_CHARGE = 1.602176634e-19
EPS0 = 8.8541878128e-12
COUL = E_CHARGE * E_CHARGE / (4.0 * math.pi * EPS0 * 1e-10)

NODE_BLK = 1000
EDGE_BLK = 3200

def _softplus(v):
    return jnp.maximum(v, 0.0) + jnp.log1p(jnp.exp(-jnp.abs(v)))


def _mish(v):
    w = jnp.exp(jnp.minimum(v, 30.0))
    u = w * (w + 2.0)
    return v * u / (u + 2.0)


def _mlp_block(v, w1, b1, w2, b2):
    t = _mish(jnp.dot(v, w1, preferred_element_type=jnp.float32) + b1)
    return jnp.dot(t, w2, preferred_element_type=jnp.float32) + b2


def _node_body(x_ref, ws1, bs1, ws2, bs2, wd1, bd1, wd2, bd2, hs_ref, hd_ref):
    xb = x_ref[...]
    hs_ref[...] = _mlp_block(xb, ws1[...], bs1[...], ws2[...], bs2[...])
    hd_ref[...] = _mlp_block(xb, wd1[...], bd1[...], wd2[...], bd2[...])


def _edge_body(ef_ref, p_ref, we1, be1, we2, be2,
               wm1, bm1, wm2, bm2, m_ref):
    ef = _mlp_block(ef_ref[...], we1[...], be1[...], we2[...], be2[...])
    hn = p_ref[...] * COUL / ef
    m_ref[...] = _mlp_block(hn, wm1[...], bm1[...], wm2[...], bm2[...])


def _final_body(x_ref, hp_ref, o_ref):
    o_ref[...] = _softplus(x_ref[...] + hp_ref[0] + hp_ref[1])


def _worker_base():
    w = lax.axis_index("s") * NC + lax.axis_index("c")
    return w * W_EDGES


_NBUF = 3
_SLOTS = W_CH + 1


def _gather_body(hs_hbm, hd_hbm, src_hbm, dst_hbm, p_hbm,
                 idx_s, idx_d, rows_s, rows_d,
                 gsem0, gsem1, gsem2, wsem0, wsem1, wsem2):
    base = _worker_base()
    gsems = (gsem0, gsem1, gsem2)
    wsems = (wsem0, wsem1, wsem2)

    def wait_wb(buf):
        pltpu.make_async_copy(
            rows_s.at[buf], p_hbm.at[pl.ds(base, CH)], wsems[buf]).wait()

    def load(c, buf):
        b = base + c * CH
        pltpu.sync_copy(src_hbm.at[pl.ds(b, CH)], idx_s.at[buf])
        pltpu.sync_copy(dst_hbm.at[pl.ds(b, CH)], idx_d.at[buf])
        pltpu.async_copy(hs_hbm.at[idx_s.at[buf]], rows_s.at[buf], gsems[buf])
        pltpu.async_copy(hd_hbm.at[idx_d.at[buf]], rows_d.at[buf], gsems[buf])

    def process(c, buf):
        pltpu.make_async_copy(
            hs_hbm.at[idx_s.at[buf]], rows_s.at[buf], gsems[buf]).wait()
        pltpu.make_async_copy(
            hd_hbm.at[idx_d.at[buf]], rows_d.at[buf], gsems[buf]).wait()

        def mul_row(i, carry):
            for j in range(D // 16):
                sl = pl.ds(j * 16, 16)
                rows_s[buf, i, sl] = rows_s[buf, i, sl] * rows_d[buf, i, sl]
            return carry

        lax.fori_loop(0, CH, mul_row, 0)
        pltpu.async_copy(rows_s.at[buf], p_hbm.at[pl.ds(base + c * CH, CH)],
                         wsems[buf])

    def triple(t, carry):
        for k in range(_NBUF):
            s = _NBUF * t + k

            @pl.when(s < W_CH)
            def _():
                @pl.when(s >= _NBUF)
                def _():
                    wait_wb(k)
                load(s, k)

            @pl.when(s >= 1)
            def _():
                process(s - 1, (k - 1) % _NBUF)
        return carry

    lax.fori_loop(0, _SLOTS // _NBUF, triple, 0)
    wait_wb((W_CH - 3) % _NBUF)
    wait_wb((W_CH - 2) % _NBUF)
    wait_wb((W_CH - 1) % _NBUF)


_BLK_R = 80
_NBLK_R = N // _BLK_R
_RB_BASE = _NBLK_R // NS
_RB_EXTRA = _NBLK_R - _RB_BASE * NS


def _scatter_body(m_hbm, dst_hbm, hp_hbm, acc, idx_v, rows_v, zrow,
                  lsem0, lsem1):
    cid = lax.axis_index("c")
    sid = lax.axis_index("s")
    base = _worker_base()
    lsems = (lsem0, lsem1)
    nblk = _RB_BASE + jnp.where(sid < _RB_EXTRA, 1, 0)

    zv = jnp.zeros((16,), jnp.float32)

    def zfill(i, carry):
        for j in range(D // 16):
            zrow[i, pl.ds(j * 16, 16)] = zv
        return carry

    lax.fori_loop(0, _BLK_R, zfill, 0)

    def zblk(c, carry):
        k = sid + NS * c
        pltpu.sync_copy(zrow, acc.at[pl.ds(k * _BLK_R, _BLK_R)])
        return carry

    lax.fori_loop(0, nblk, zblk, 0)
    plsc.subcore_barrier()

    def load(c, buf):
        b = base + c * CH
        pltpu.async_copy(dst_hbm.at[pl.ds(b, CH)], idx_v.at[buf], lsems[buf])
        pltpu.async_copy(m_hbm.at[pl.ds(b, CH)], rows_v.at[buf], lsems[buf])

    def flush(c, buf):
        b = base + c * CH
        pltpu.make_async_copy(
            dst_hbm.at[pl.ds(b, CH)], idx_v.at[buf], lsems[buf]).wait()
        pltpu.make_async_copy(
            m_hbm.at[pl.ds(b, CH)], rows_v.at[buf], lsems[buf]).wait()
        pltpu.sync_copy(rows_v.at[buf], acc.at[idx_v.at[buf]], add=True)

    load(0, 0)

    def pair(p, carry):
        load(2 * p + 1, 1)
        flush(2 * p, 0)
        load(2 * p + 2, 0)
        flush(2 * p + 1, 1)
        return carry

    lax.fori_loop(0, W_PAIRS, pair, 0)
    flush(W_CH - 1, 0)
    plsc.subcore_barrier()

    def wblk(c, carry):
        k = sid + NS * c
        pltpu.sync_copy(acc.at[pl.ds(k * _BLK_R, _BLK_R)],
                        hp_hbm.at[cid, pl.ds(k * _BLK_R, _BLK_R)])
        return carry

    lax.fori_loop(0, nblk, wblk, 0)


@functools.cache
def _sc_kernels():
    mesh = plsc.VectorSubcoreMesh(
        core_axis_name="c", subcore_axis_name="s",
        num_cores=NC, num_subcores=NS)
    gather = pl.kernel(
        _gather_body,
        out_type=jax.ShapeDtypeStruct((E, D), jnp.float32),
        mesh=mesh,
        scratch_types=[
            pltpu.VMEM((_NBUF, CH), jnp.int32),
            pltpu.VMEM((_NBUF, CH), jnp.int32),
            pltpu.VMEM((_NBUF, CH, D), jnp.float32),
            pltpu.VMEM((_NBUF, CH, D), jnp.float32),
        ] + [pltpu.SemaphoreType.DMA] * 6,
    )
    scatter = pl.kernel(
        _scatter_body,
        out_type=jax.ShapeDtypeStruct((NC, N, D), jnp.float32),
        mesh=mesh,
        scratch_types=[
            pltpu.VMEM_SHARED((N, D), jnp.float32),
            pltpu.VMEM((2, CH), jnp.int32),
            pltpu.VMEM((2, CH, D), jnp.float32),
            pltpu.VMEM((_BLK_R, D), jnp.float32),
        ] + [pltpu.SemaphoreType.DMA] * 2,
    )
    return gather, scatter


def _rep(i):
    return (0, 0)


def kernel(x, edge_index, edge_feats,
           W_src_1, b_src_1, W_src_2, b_src_2,
           W_dst_1, b_dst_1, W_dst_2, b_dst_2,
           W_edge_1, b_edge_1, W_edge_2, b_edge_2,
           W_m_1, b_m_1, W_m_2, b_m_2):
    src = edge_index[0]
    dst = edge_index[1]
    biases = [b.reshape(1, D) for b in
              (b_src_1, b_src_2, b_dst_1, b_dst_2,
               b_edge_1, b_edge_2, b_m_1, b_m_2)]
    bs1, bs2, bd1, bd2, be1, be2, bm1, bm2 = biases

    wspec = pl.BlockSpec((D, D), _rep)
    bspec = pl.BlockSpec((1, D), _rep)

    h_src, h_dst = pl.pallas_call(
        _node_body,
        grid=(N // NODE_BLK,),
        in_specs=[pl.BlockSpec((NODE_BLK, D), lambda i: (i, 0)),
                  wspec, bspec, wspec, bspec,
                  wspec, bspec, wspec, bspec],
        out_specs=[pl.BlockSpec((NODE_BLK, D), lambda i: (i, 0))] * 2,
        out_shape=[jax.ShapeDtypeStruct((N, D), jnp.float32)] * 2,
    )(x, W_src_1, bs1, W_src_2, bs2, W_dst_1, bd1, W_dst_2, bd2)

    gather_k, scatter_k = _sc_kernels()
    p = gather_k(h_src, h_dst, src, dst)

    m = pl.pallas_call(
        _edge_body,
        grid=(E // EDGE_BLK,),
        in_specs=[pl.BlockSpec((EDGE_BLK, D), lambda i: (i, 0))] * 2 +
                 [wspec, bspec, wspec, bspec, wspec, bspec, wspec, bspec],
        out_specs=pl.BlockSpec((EDGE_BLK, D), lambda i: (i, 0)),
        out_shape=jax.ShapeDtypeStruct((E, D), jnp.float32),
    )(edge_feats, p, W_edge_1, be1, W_edge_2, be2,
      W_m_1, bm1, W_m_2, bm2)

    hp = scatter_k(m, dst)

    out = pl.pallas_call(
        _final_body,
        grid=(N // NODE_BLK,),
        in_specs=[pl.BlockSpec((NODE_BLK, D), lambda i: (i, 0)),
                  pl.BlockSpec((NC, NODE_BLK, D), lambda i: (0, i, 0))],
        out_specs=pl.BlockSpec((NODE_BLK, D), lambda i: (i, 0)),
        out_shape=jax.ShapeDtypeStruct((N, D), jnp.float32),
    )(x, hp)
    return out


# speedup vs baseline: 5.4036x; 1.2243x over previous
import functools
import math

import jax
import jax.numpy as jnp
from jax import lax
from jax.experimental import pallas as pl
from jax.experimental.pallas import tpu as pltpu
from jax.experimental.pallas import tpu_sc as plsc

N = 10000
E = 320000
D = 128

NC = 2
NS = 16
NW = NC * NS

NSPLIT = 2
ES = E // NSPLIT
CH = 128
ES_CH = ES // CH
W_BASE = ES_CH // NW
W_EXTRA = ES_CH - W_BASE * NW

E_CHARGE = 1.602176634e-19
EPS0 = 8.8541878128e-12
COUL = E_CHARGE * E_CHARGE / (4.0 * math.pi * EPS0 * 1e-10)

NODE_BLK = 1000
EDGE_BLK = 3200


def _softplus(v):
    return jnp.maximum(v, 0.0) + jnp.log1p(jnp.exp(-jnp.abs(v)))


def _mish(v):
    w = jnp.exp(jnp.minimum(v, 30.0))
    u = w * (w + 2.0)
    return v * u / (u + 2.0)


def _dot(a, w):
    return jnp.dot(a.astype(jnp.bfloat16), w.astype(jnp.bfloat16),
                   preferred_element_type=jnp.float32)


def _mlp_block(v, w1, b1, w2, b2):
    t = _mish(_dot(v, w1) + b1)
    return _dot(t, w2) + b2


def _node_body(x_ref, ws1, bs1, ws2, bs2, wd1, bd1, wd2, bd2, hs_ref, hd_ref):
    xb = x_ref[...]
    hs_ref[...] = _mlp_block(xb, ws1[...], bs1[...], ws2[...], bs2[...])
    hd_ref[...] = _mlp_block(xb, wd1[...], bd1[...], wd2[...], bd2[...])


def _edge_body(ef_ref, p_ref, we1, be1, we2, be2,
               wm1, bm1, wm2, bm2, m_ref):
    ef = _mlp_block(ef_ref[...], we1[...], be1[...], we2[...], be2[...])
    hn = p_ref[...].astype(jnp.float32) * COUL / ef
    m_ref[...] = _mlp_block(hn, wm1[...], bm1[...], wm2[...], bm2[...])


def _final_body(x_ref, *refs):
    o_ref = refs[-1]
    acc = x_ref[...]
    for h in refs[:-1]:
        acc = acc + h[0] + h[1]
    o_ref[...] = _softplus(acc)


def _worker_sched():
    w = lax.axis_index("s") * NC + lax.axis_index("c")
    nch = W_BASE + jnp.where(w < W_EXTRA, 1, 0)
    return w, nch


def _chunk_base(w, c):
    return (w + NW * c) * CH


_NBUF = 3
DW = D // 2


def _gather_body(soff, hs_hbm, hd_hbm, src_hbm, dst_hbm, p_hbm,
                 idx_s, idx_d, rows_s, rows_d,
                 gsem0, gsem1, gsem2, wsem0, wsem1, wsem2):
    w, nch = _worker_sched()
    gsems = (gsem0, gsem1, gsem2)
    wsems = (wsem0, wsem1, wsem2)

    def wait_wb(buf):
        pltpu.make_async_copy(
            rows_s.at[buf], p_hbm.at[pl.ds(0, CH)], wsems[buf]).wait()

    def load(c, buf):
        b = _chunk_base(w, c)
        pltpu.sync_copy(src_hbm.at[pl.ds(soff + b, CH)], idx_s.at[buf])
        pltpu.sync_copy(dst_hbm.at[pl.ds(soff + b, CH)], idx_d.at[buf])
        pltpu.async_copy(hs_hbm.at[idx_s.at[buf]], rows_s.at[buf], gsems[buf])
        pltpu.async_copy(hd_hbm.at[idx_d.at[buf]], rows_d.at[buf], gsems[buf])

    def process(c, buf):
        pltpu.make_async_copy(
            hs_hbm.at[idx_s.at[buf]], rows_s.at[buf], gsems[buf]).wait()
        pltpu.make_async_copy(
            hd_hbm.at[idx_d.at[buf]], rows_d.at[buf], gsems[buf]).wait()

        def mul_row(i, carry):
            for j in range(D // 16):
                sl = pl.ds(j * 16, 16)
                rows_s[buf, i, sl] = rows_s[buf, i, sl] * rows_d[buf, i, sl]
            return carry

        lax.fori_loop(0, CH, mul_row, 0)
        pltpu.async_copy(rows_s.at[buf], p_hbm.at[pl.ds(_chunk_base(w, c), CH)],
                         wsems[buf])

    def triple(t, carry):
        for k in range(_NBUF):
            s = _NBUF * t + k

            @pl.when(s < nch)
            def _():
                @pl.when(s >= _NBUF)
                def _():
                    wait_wb(k)
                load(s, k)

            @pl.when(jnp.logical_and(s >= 1, s <= nch))
            def _():
                process(s - 1, (k - 1) % _NBUF)
        return carry

    lax.fori_loop(0, (nch + _NBUF) // _NBUF, triple, 0)
    wait_wb(0)
    wait_wb(1)
    wait_wb(2)


_BLK_R = 80
_NBLK_R = N // _BLK_R
_RB_BASE = _NBLK_R // NS
_RB_EXTRA = _NBLK_R - _RB_BASE * NS


def _scatter_body(soff, m_hbm, dst_hbm, hp_hbm, acc, idx_v, rows_v, zrow,
                  lsem0, lsem1):
    cid = lax.axis_index("c")
    sid = lax.axis_index("s")
    w, nch = _worker_sched()
    lsems = (lsem0, lsem1)
    nblk = _RB_BASE + jnp.where(sid < _RB_EXTRA, 1, 0)

    zv = jnp.zeros((16,), jnp.float32)

    def zfill(i, carry):
        for j in range(D // 16):
            zrow[i, pl.ds(j * 16, 16)] = zv
        return carry

    lax.fori_loop(0, _BLK_R, zfill, 0)

    def zblk(c, carry):
        k = sid + NS * c
        pltpu.sync_copy(zrow, acc.at[pl.ds(k * _BLK_R, _BLK_R)])
        return carry

    lax.fori_loop(0, nblk, zblk, 0)
    plsc.subcore_barrier()

    def load(c, buf):
        b = _chunk_base(w, c)
        pltpu.async_copy(dst_hbm.at[pl.ds(soff + b, CH)], idx_v.at[buf],
                         lsems[buf])
        pltpu.async_copy(m_hbm.at[pl.ds(b, CH)], rows_v.at[buf], lsems[buf])

    def flush(c, buf):
        b = _chunk_base(w, c)
        pltpu.make_async_copy(
            dst_hbm.at[pl.ds(soff + b, CH)], idx_v.at[buf], lsems[buf]).wait()
        pltpu.make_async_copy(
            m_hbm.at[pl.ds(b, CH)], rows_v.at[buf], lsems[buf]).wait()
        pltpu.sync_copy(rows_v.at[buf], acc.at[idx_v.at[buf]], add=True)

    def duo(t, carry):
        for k in range(2):
            s = 2 * t + k

            @pl.when(s < nch)
            def _():
                load(s, k)

            @pl.when(jnp.logical_and(s >= 1, s <= nch))
            def _():
                flush(s - 1, (k - 1) % 2)
        return carry

    lax.fori_loop(0, (nch + 2) // 2, duo, 0)
    plsc.subcore_barrier()

    def wblk(c, carry):
        k = sid + NS * c
        pltpu.sync_copy(acc.at[pl.ds(k * _BLK_R, _BLK_R)],
                        hp_hbm.at[cid, pl.ds(k * _BLK_R, _BLK_R)])
        return carry

    lax.fori_loop(0, nblk, wblk, 0)


@functools.cache
def _sc_kernels(soff):
    mesh = plsc.VectorSubcoreMesh(
        core_axis_name="c", subcore_axis_name="s",
        num_cores=NC, num_subcores=NS)
    gather = pl.kernel(
        functools.partial(_gather_body, soff),
        out_type=jax.ShapeDtypeStruct((ES, D), jnp.float32),
        mesh=mesh,
        scratch_types=[
            pltpu.VMEM((_NBUF, CH), jnp.int32),
            pltpu.VMEM((_NBUF, CH), jnp.int32),
            pltpu.VMEM((_NBUF, CH, D), jnp.float32),
            pltpu.VMEM((_NBUF, CH, D), jnp.float32),
        ] + [pltpu.SemaphoreType.DMA] * 6,
    )
    scatter = pl.kernel(
        functools.partial(_scatter_body, soff),
        out_type=jax.ShapeDtypeStruct((NC, N, D), jnp.float32),
        mesh=mesh,
        scratch_types=[
            pltpu.VMEM_SHARED((N, D), jnp.float32),
            pltpu.VMEM((2, CH), jnp.int32),
            pltpu.VMEM((2, CH, D), jnp.float32),
            pltpu.VMEM((_BLK_R, D), jnp.float32),
        ] + [pltpu.SemaphoreType.DMA] * 2,
    )
    return gather, scatter


def _rep(i):
    return (0, 0)


def kernel(x, edge_index, edge_feats,
           W_src_1, b_src_1, W_src_2, b_src_2,
           W_dst_1, b_dst_1, W_dst_2, b_dst_2,
           W_edge_1, b_edge_1, W_edge_2, b_edge_2,
           W_m_1, b_m_1, W_m_2, b_m_2):
    src = edge_index[0]
    dst = edge_index[1]
    biases = [b.reshape(1, D) for b in
              (b_src_1, b_src_2, b_dst_1, b_dst_2,
               b_edge_1, b_edge_2, b_m_1, b_m_2)]
    bs1, bs2, bd1, bd2, be1, be2, bm1, bm2 = biases

    wspec = pl.BlockSpec((D, D), _rep)
    bspec = pl.BlockSpec((1, D), _rep)

    h_src, h_dst = pl.pallas_call(
        _node_body,
        grid=(N // NODE_BLK,),
        in_specs=[pl.BlockSpec((NODE_BLK, D), lambda i: (i, 0)),
                  wspec, bspec, wspec, bspec,
                  wspec, bspec, wspec, bspec],
        out_specs=[pl.BlockSpec((NODE_BLK, D), lambda i: (i, 0))] * 2,
        out_shape=[jax.ShapeDtypeStruct((N, D), jnp.float32)] * 2,
    )(x, W_src_1, bs1, W_src_2, bs2, W_dst_1, bd1, W_dst_2, bd2)

    hps = []
    for si in range(NSPLIT):
        gather_k, scatter_k = _sc_kernels(si * ES)
        goff = si * (ES // EDGE_BLK)
        p = gather_k(h_src, h_dst, src, dst)
        m = pl.pallas_call(
            _edge_body,
            grid=(ES // EDGE_BLK,),
            in_specs=[
                pl.BlockSpec((EDGE_BLK, D),
                             functools.partial(lambda g, i: (g + i, 0), goff)),
                pl.BlockSpec((EDGE_BLK, D), lambda i: (i, 0)),
                wspec, bspec, wspec, bspec, wspec, bspec, wspec, bspec],
            out_specs=pl.BlockSpec((EDGE_BLK, D), lambda i: (i, 0)),
            out_shape=jax.ShapeDtypeStruct((ES, D), jnp.float32),
        )(edge_feats, p, W_edge_1, be1, W_edge_2, be2,
          W_m_1, bm1, W_m_2, bm2)
        hps.append(scatter_k(m, dst))

    out = pl.pallas_call(
        _final_body,
        grid=(N // NODE_BLK,),
        in_specs=[pl.BlockSpec((NODE_BLK, D), lambda i: (i, 0))] +
                 [pl.BlockSpec((NC, NODE_BLK, D), lambda i: (0, i, 0))] * NSPLIT,
        out_specs=pl.BlockSpec((NODE_BLK, D), lambda i: (i, 0)),
        out_shape=jax.ShapeDtypeStruct((N, D), jnp.float32),
    )(x, *hps)
    return out
